# baseline (speedup 1.0000x reference)
import functools

import jax
import jax.numpy as jnp
from jax import lax
from jax.experimental import pallas as pl
from jax.experimental.pallas import tpu as pltpu
from jax.experimental.pallas import tpu_sc as plsc

N_NODES = 10000
D_FEAT = 128
N_EDGES = 320000

NC = 2
NS = 16
N_TILES = NC * NS

EDGES_PER_TILE = N_EDGES // N_TILES
CHUNK = 50
ITERS = EDGES_PER_TILE // CHUNK
RING = 4
GRP = (ITERS - 4) // RING
N_PAD = N_NODES
ROWS_PER_TILE = N_PAD // NS


def _sc_scatter_add(x, edge3, zeros):
  mesh = plsc.VectorSubcoreMesh(core_axis_name="c", subcore_axis_name="s")

  @functools.partial(
      pl.kernel,
      mesh=mesh,
      compiler_params=pltpu.CompilerParams(use_tc_tiling_on_sc=False),
      out_type=jax.ShapeDtypeStruct((NC, N_PAD, D_FEAT), jnp.float32),
      scratch_types=[
          pltpu.VMEM((ITERS, CHUNK), jnp.int32),
          pltpu.VMEM((ITERS, CHUNK), jnp.int32),
          [pltpu.VMEM((CHUNK, D_FEAT), jnp.float32) for _ in range(RING)],
          pltpu.VMEM_SHARED((N_PAD, D_FEAT), jnp.float32),
          [pltpu.SemaphoreType.DMA for _ in range(RING)],
          [pltpu.SemaphoreType.DMA for _ in range(RING)],
      ],
  )
  def body(x_hbm, e_hbm, zeros_hbm, out_hbm, sidx, didx, rows, ysh, gsem, ssem):
    c = lax.axis_index("c")
    s = lax.axis_index("s")
    wid = c * NS + s
    row0 = pl.multiple_of(s * ROWS_PER_TILE, 8)
    pltpu.sync_copy(zeros_hbm, ysh.at[pl.ds(row0, ROWS_PER_TILE)])
    pltpu.sync_copy(e_hbm.at[0].at[wid], sidx)
    pltpu.sync_copy(e_hbm.at[1].at[wid], didx)

    def fire_g(i, j):
      pltpu.async_copy(x_hbm.at[sidx.at[i]], rows[j], gsem[j])

    def wait_g(i, j):
      pltpu.make_async_copy(x_hbm.at[sidx.at[i]], rows[j], gsem[j]).wait()

    def fire_s(i, j):
      pltpu.async_copy(rows[j], ysh.at[didx.at[i]], ssem[j], add=True)

    def wait_s(i, j):
      pltpu.make_async_copy(rows[j], ysh.at[didx.at[i]], ssem[j]).wait()

    fire_g(0, 0)
    fire_g(1, 1)
    plsc.subcore_barrier()

    wait_g(0, 0)
    fire_s(0, 0)
    fire_g(2, 2)
    wait_g(1, 1)
    fire_s(1, 1)
    fire_g(3, 3)

    def step(i, sj):
      wait_g(i, sj)
      fire_s(i, sj)
      wait_s(i - 2, (sj + 2) % RING)
      fire_g(i + 2, (sj + 2) % RING)

    def group(g, carry):
      i0 = 2 + RING * g
      for j in range(RING):
        step(i0 + j, (2 + j) % RING)
      return carry

    lax.fori_loop(0, GRP, group, 0)

    i0 = ITERS - 2
    wait_g(i0, i0 % RING)
    fire_s(i0, i0 % RING)
    wait_s(i0 - 2, (i0 + 2) % RING)
    wait_g(i0 + 1, (i0 + 1) % RING)
    fire_s(i0 + 1, (i0 + 1) % RING)
    wait_s(i0 - 1, (i0 + 3) % RING)
    wait_s(i0, i0 % RING)
    wait_s(i0 + 1, (i0 + 1) % RING)

    plsc.subcore_barrier()
    pltpu.sync_copy(ysh.at[pl.ds(row0, ROWS_PER_TILE)],
                    out_hbm.at[c].at[pl.ds(row0, ROWS_PER_TILE)])

  return body(x, edge3, zeros)


def _tc_linear(y2, W, b):
  BM = 2000

  def body(y_ref, w_ref, b_ref, o_ref):
    ysum = y_ref[0] + y_ref[1]
    o_ref[...] = lax.dot_general(
        ysum, w_ref[...], (((1,), (1,)), ((), ())),
        preferred_element_type=jnp.float32) + b_ref[...]

  return pl.pallas_call(
      body,
      grid=(N_NODES // BM,),
      in_specs=[
          pl.BlockSpec((NC, BM, D_FEAT), lambda i: (0, i, 0)),
          pl.BlockSpec((D_FEAT, D_FEAT), lambda i: (0, 0)),
          pl.BlockSpec((1, D_FEAT), lambda i: (0, 0)),
      ],
      out_specs=pl.BlockSpec((BM, D_FEAT), lambda i: (i, 0)),
      out_shape=jax.ShapeDtypeStruct((N_NODES, D_FEAT), jnp.float32),
  )(y2, W, b.reshape(1, D_FEAT))


def kernel(x, edge_index, W, b):
  if edge_index.dtype != jnp.int32:
    edge_index = edge_index.astype(jnp.int32)
  edge3 = edge_index.reshape(2, N_TILES, ITERS, CHUNK)
  zeros = jnp.zeros((ROWS_PER_TILE, D_FEAT), jnp.float32)
  y2 = _sc_scatter_add(x, edge3, zeros)
  return _tc_linear(y2, W, b)


# speedup vs baseline: 1.4012x; 1.4012x over previous
import functools

import jax
import jax.numpy as jnp
from jax import lax
from jax.experimental import pallas as pl
from jax.experimental.pallas import tpu as pltpu
from jax.experimental.pallas import tpu_sc as plsc

N_NODES = 10000
D_FEAT = 128
N_EDGES = 320000

NC = 2
NS = 16
N_TILES = NC * NS

EDGES_PER_TILE = N_EDGES // N_TILES
CHUNK = 80
ITERS = EDGES_PER_TILE // CHUNK
RING = 3
GRP = (ITERS - 2) // RING - 1
N_PAD = N_NODES
ROWS_PER_TILE = N_PAD // NS


def _sc_scatter_add(x, edge3, zeros):
  mesh = plsc.VectorSubcoreMesh(core_axis_name="c", subcore_axis_name="s")

  @functools.partial(
      pl.kernel,
      mesh=mesh,
      compiler_params=pltpu.CompilerParams(use_tc_tiling_on_sc=False),
      out_type=jax.ShapeDtypeStruct((NC, N_PAD, D_FEAT), jnp.float32),
      scratch_types=[
          pltpu.VMEM((ITERS, CHUNK), jnp.int32),
          pltpu.VMEM((ITERS, CHUNK), jnp.int32),
          [pltpu.VMEM((CHUNK, D_FEAT), jnp.float32) for _ in range(RING)],
          pltpu.VMEM_SHARED((N_PAD, D_FEAT), jnp.float32),
          [pltpu.SemaphoreType.DMA for _ in range(RING)],
          [pltpu.SemaphoreType.DMA for _ in range(RING)],
      ],
  )
  def body(x_hbm, e_hbm, zeros_hbm, out_hbm, sidx, didx, rows, ysh, gsem, ssem):
    c = lax.axis_index("c")
    s = lax.axis_index("s")
    wid = c * NS + s
    row0 = pl.multiple_of(s * ROWS_PER_TILE, 8)
    pltpu.sync_copy(zeros_hbm, ysh.at[pl.ds(row0, ROWS_PER_TILE)])
    pltpu.sync_copy(e_hbm.at[0].at[wid], sidx)
    pltpu.sync_copy(e_hbm.at[1].at[wid], didx)

    def fire_g(i, j):
      pltpu.async_copy(x_hbm.at[sidx.at[i]], rows[j], gsem[j])

    def wait_g(i, j):
      pltpu.make_async_copy(x_hbm.at[sidx.at[i]], rows[j], gsem[j]).wait()

    def fire_s(i, j):
      pltpu.async_copy(rows[j], ysh.at[didx.at[i]], ssem[j], add=True)

    def wait_s(i, j):
      pltpu.make_async_copy(rows[j], ysh.at[didx.at[i]], ssem[j]).wait()

    fire_g(0, 0)
    fire_g(1, 1)
    plsc.subcore_barrier()

    wait_g(0, 0)
    fire_s(0, 0)
    fire_g(2, 2)
    wait_g(1, 1)
    fire_s(1, 1)
    wait_s(0, 0)
    fire_g(3, 0)

    def step(i, sj):
      wait_g(i, sj)
      fire_s(i, sj)
      wait_s(i - 1, (sj + 2) % RING)
      fire_g(i + 2, (sj + 2) % RING)

    def group(g, carry):
      i0 = 2 + RING * g
      for j in range(RING):
        step(i0 + j, (2 + j) % RING)
      return carry

    lax.fori_loop(0, GRP, group, 0)

    i0 = ITERS - RING
    wait_g(i0, i0 % RING)
    fire_s(i0, i0 % RING)
    wait_s(i0 - 1, (i0 + 2) % RING)
    fire_g(i0 + 2, (i0 + 2) % RING)
    wait_g(i0 + 1, (i0 + 1) % RING)
    fire_s(i0 + 1, (i0 + 1) % RING)
    wait_s(i0, i0 % RING)
    wait_g(i0 + 2, (i0 + 2) % RING)
    fire_s(i0 + 2, (i0 + 2) % RING)
    wait_s(i0 + 1, (i0 + 1) % RING)
    wait_s(i0 + 2, (i0 + 2) % RING)

    plsc.subcore_barrier()
    pltpu.sync_copy(ysh.at[pl.ds(row0, ROWS_PER_TILE)],
                    out_hbm.at[c].at[pl.ds(row0, ROWS_PER_TILE)])

  return body(x, edge3, zeros)


def _tc_linear(y2, W, b):
  BM = 2000

  def body(y_ref, w_ref, b_ref, o_ref):
    ysum = y_ref[0] + y_ref[1]
    o_ref[...] = lax.dot_general(
        ysum, w_ref[...], (((1,), (1,)), ((), ())),
        preferred_element_type=jnp.float32) + b_ref[...]

  return pl.pallas_call(
      body,
      grid=(N_NODES // BM,),
      in_specs=[
          pl.BlockSpec((NC, BM, D_FEAT), lambda i: (0, i, 0)),
          pl.BlockSpec((D_FEAT, D_FEAT), lambda i: (0, 0)),
          pl.BlockSpec((1, D_FEAT), lambda i: (0, 0)),
      ],
      out_specs=pl.BlockSpec((BM, D_FEAT), lambda i: (i, 0)),
      out_shape=jax.ShapeDtypeStruct((N_NODES, D_FEAT), jnp.float32),
  )(y2, W, b.reshape(1, D_FEAT))


def kernel(x, edge_index, W, b):
  if edge_index.dtype != jnp.int32:
    edge_index = edge_index.astype(jnp.int32)
  edge3 = edge_index.reshape(2, N_TILES, ITERS, CHUNK)
  zeros = jnp.zeros((ROWS_PER_TILE, D_FEAT), jnp.float32)
  y2 = _sc_scatter_add(x, edge3, zeros)
  return _tc_linear(y2, W, b)
